# baseline (speedup 1.0000x reference)
import functools

import jax
import jax.numpy as jnp
from jax import lax
from jax.experimental import pallas as pl
from jax.experimental.pallas import tpu as pltpu
from jax.experimental.pallas import tpu_sc as plsc

B = 16384
VOCAB1 = 1000001
DIM = 32
ODIM = 2 * DIM + 1
NBUCKETS = 1000
TSROWS = 1024
L = 16

_NC = 2
_NS = 16
_NW = _NC * _NS
_BPW = B // _NW
_NBUF = 8
_PITCH = 72

_STEPS = (512, 256, 128, 64, 32, 16, 8, 4, 2, 1)


def _body(uid_hbm, ts_hbm, utab_hbm, ttab_hbm, bkt_hbm, mean_hbm, scale_hbm,
          out_hbm, idx_v, ts_v, tsj_v, ring_v, ttab_v, out_v, bkt_v, ms_v,
          sems, sem_t):
    wid = lax.axis_index("s") * _NC + lax.axis_index("c")
    base = wid * _BPW
    lane = lax.iota(jnp.int32, L)

    pltpu.sync_copy(uid_hbm.at[pl.ds(base, _BPW)], idx_v.at[pl.ds(0, _BPW)])
    pltpu.sync_copy(ts_hbm.at[pl.ds(base, _BPW)], ts_v)
    pltpu.sync_copy(bkt_hbm, bkt_v)
    pltpu.sync_copy(mean_hbm, ms_v.at[pl.ds(0, L)])
    pltpu.sync_copy(scale_hbm, ms_v.at[pl.ds(L, L)])
    cp_tt = pltpu.make_async_copy(ttab_hbm, ttab_v, sem_t)
    cp_tt.start()

    def _fetch(r, slot, h):
        uvec = idx_v[pl.ds(r, L)]
        b = uvec[0] >> 7
        return pltpu.make_async_copy(
            utab_hbm.at[pl.ds(h * L, L), pl.ds(b * 128, 128)],
            ring_v.at[2 * slot + h], sems.at[2 * slot + h])

    for s in range(_NBUF):
        _fetch(s, s, 0).start()
        _fetch(s, s, 1).start()

    mean = ms_v[pl.ds(0, L)]
    scale = ms_v[pl.ds(L, L)]

    def bucketize(i, carry):
        off = pl.multiple_of(i * L, L)
        t = ts_v[pl.ds(off, L)]
        pos = jnp.zeros((L,), jnp.int32)
        for step in _STEPS:
            cand = pos + step
            safe = jnp.minimum(cand - 1, NBUCKETS - 1)
            bv = plsc.load_gather(bkt_v, [safe])
            take = jnp.logical_and(cand <= NBUCKETS, bv < t)
            pos = jnp.where(take, cand, pos)
        tsj_v[pl.ds(off, L)] = pos * DIM
        rows = (off + lane) * _PITCH + 2 * DIM
        plsc.store_scatter(out_v, [rows], (t - mean) * scale)
        return carry

    def ublock(g, carry):
        @pl.when(g & 1 == 0)
        def _():
            bucketize(g >> 1, 0)

        r0 = pl.multiple_of(g * _NBUF, _NBUF)
        for s in range(_NBUF):
            r = r0 + s
            uvec = idx_v[pl.ds(r, L)]
            c = jnp.full((L,), uvec[0] & 127, jnp.int32)
            ro = pl.multiple_of(r * _PITCH, 8)
            _fetch(r, s, 0).wait()
            out_v[pl.ds(ro, L)] = plsc.load_gather(ring_v.at[2 * s],
                                                   [lane, c])
            _fetch(r, s, 1).wait()
            out_v[pl.ds(ro + L, L)] = plsc.load_gather(ring_v.at[2 * s + 1],
                                                       [lane, c])

            @pl.when(r + _NBUF < _BPW)
            def _():
                _fetch(r + _NBUF, s, 0).start()
                _fetch(r + _NBUF, s, 1).start()
        return carry

    lax.fori_loop(0, _BPW // _NBUF, ublock, 0)

    cp_tt.wait()

    def tsblock(i, carry):
        off = pl.multiple_of(i * L, L)
        jvec = tsj_v[pl.ds(off, L)]
        for s in range(L):
            joff = pl.multiple_of(jvec[s], DIM)
            ro = pl.multiple_of((off + s) * _PITCH, 8)
            out_v[pl.ds(ro + 2 * L, L)] = ttab_v[pl.ds(joff, L)]
            out_v[pl.ds(ro + 3 * L, L)] = ttab_v[pl.ds(joff + L, L)]
        return carry

    lax.fori_loop(0, _BPW // L, tsblock, 0)

    pltpu.sync_copy(out_v, out_hbm.at[pl.ds(base * _PITCH, _BPW * _PITCH)])


@jax.jit
def _run(user_id, timestamp, utab_t, ttab_f, buckets, mean16, scale16):
    mesh = plsc.VectorSubcoreMesh(core_axis_name="c", subcore_axis_name="s")
    f = functools.partial(
        pl.kernel,
        mesh=mesh,
        out_type=jax.ShapeDtypeStruct((B * _PITCH,), jnp.float32),
        scratch_types=[
            pltpu.VMEM((_BPW + L,), jnp.int32),
            pltpu.VMEM((_BPW,), jnp.float32),
            pltpu.VMEM((_BPW,), jnp.int32),
            pltpu.VMEM((2 * _NBUF, L, 128), jnp.float32),
            pltpu.VMEM((TSROWS * DIM,), jnp.float32),
            pltpu.VMEM((_BPW * _PITCH,), jnp.float32),
            pltpu.VMEM((NBUCKETS,), jnp.float32),
            pltpu.VMEM((2 * L,), jnp.float32),
            pltpu.SemaphoreType.DMA((2 * _NBUF,)),
            pltpu.SemaphoreType.DMA,
        ],
        compiler_params=pltpu.CompilerParams(use_tc_tiling_on_sc=True,
                                             needs_layout_passes=False,
                                             disable_bounds_checks=True),
    )(_body)
    out = f(user_id, timestamp, utab_t, ttab_f, buckets, mean16, scale16)
    return out.reshape(B, _PITCH)[:, :ODIM]


def kernel(user_id, timestamp, user_table, ts_table, buckets, norm_mean,
           norm_var):
    scale = lax.rsqrt(norm_var[0] + 1e-6)
    mean16 = jnp.broadcast_to(norm_mean[0], (L,))
    scale16 = jnp.broadcast_to(scale, (L,))
    utab_t = user_table.T
    ttab_f = jnp.pad(ts_table.reshape(-1),
                     (0, (TSROWS - ts_table.shape[0]) * DIM))
    return _run(user_id.astype(jnp.int32), timestamp, utab_t, ttab_f,
                buckets, mean16, scale16)


# speedup vs baseline: 1.2966x; 1.2966x over previous
import functools

import jax
import jax.numpy as jnp
from jax import lax
from jax.experimental import pallas as pl
from jax.experimental.pallas import tpu as pltpu
from jax.experimental.pallas import tpu_sc as plsc

B = 16384
VOCAB1 = 1000001
DIM = 32
ODIM = 2 * DIM + 1
NBUCKETS = 1000
TSROWS = 1024
L = 16

_NC = 2
_NS = 16
_NW = _NC * _NS
_BPW = B // _NW
_NBUF = 8
_PITCH = 72

_STEPS = (512, 256, 128, 64, 32, 16, 8, 4, 2, 1)


def _body(uid_hbm, ts_hbm, utab_hbm, ttab_hbm, bkt_hbm, mean_hbm, scale_hbm,
          out_hbm, idx_v, ts_v, tsj_v, ring_v, ttab_v, out_v, bkt_v, ms_v,
          sems, sem_t):
    wid = lax.axis_index("s") * _NC + lax.axis_index("c")
    base = wid * _BPW
    lane = lax.iota(jnp.int32, L)

    pltpu.sync_copy(uid_hbm.at[pl.ds(base, _BPW)], idx_v.at[pl.ds(0, _BPW)])
    pltpu.sync_copy(ts_hbm.at[pl.ds(base, _BPW)], ts_v)
    pltpu.sync_copy(bkt_hbm, bkt_v)
    pltpu.sync_copy(mean_hbm, ms_v.at[pl.ds(0, L)])
    pltpu.sync_copy(scale_hbm, ms_v.at[pl.ds(L, L)])
    cp_tt = pltpu.make_async_copy(ttab_hbm, ttab_v, sem_t)
    cp_tt.start()

    def _fetch(r, slot):
        uvec = idx_v[pl.ds(r, L)]
        b = uvec[0] >> 7
        return pltpu.make_async_copy(
            utab_hbm.at[:, pl.ds(b * 128, 128)], ring_v.at[slot],
            sems.at[slot])

    for s in range(_NBUF):
        _fetch(s, s).start()

    mean = ms_v[pl.ds(0, L)]
    scale = ms_v[pl.ds(L, L)]

    def bucketize(i, carry):
        off = pl.multiple_of(i * L, L)
        t = ts_v[pl.ds(off, L)]
        pos = jnp.zeros((L,), jnp.int32)
        for step in _STEPS:
            cand = pos + step
            safe = jnp.minimum(cand - 1, NBUCKETS - 1)
            bv = plsc.load_gather(bkt_v, [safe])
            take = jnp.logical_and(cand <= NBUCKETS, bv < t)
            pos = jnp.where(take, cand, pos)
        tsj_v[pl.ds(off, L)] = pos * DIM
        rows = (off + lane) * _PITCH + 2 * DIM
        plsc.store_scatter(out_v, [rows], (t - mean) * scale)
        return carry

    def tsblock(i, carry):
        off = pl.multiple_of(i * L, L)
        jvec = tsj_v[pl.ds(off, L)]
        for s in range(L):
            joff = pl.multiple_of(jvec[s], DIM)
            ro = pl.multiple_of((off + s) * _PITCH, 8)
            out_v[pl.ds(ro + 2 * L, L)] = ttab_v[pl.ds(joff, L)]
            out_v[pl.ds(ro + 3 * L, L)] = ttab_v[pl.ds(joff + L, L)]
        return carry

    cp_tt.wait()

    def ublock(g, carry):
        @pl.when(g & 1 == 0)
        def _():
            bucketize(g >> 1, 0)

        @pl.when(g & 1 == 1)
        def _():
            tsblock(g >> 1, 0)

        r0 = pl.multiple_of(g * _NBUF, _NBUF)
        for s in range(_NBUF):
            r = r0 + s
            _fetch(r, s).wait()
            uvec = idx_v[pl.ds(r, L)]
            c = jnp.full((L,), uvec[0] & 127, jnp.int32)
            ro = pl.multiple_of(r * _PITCH, 8)
            out_v[pl.ds(ro, L)] = plsc.load_gather(ring_v.at[s], [lane, c])
            out_v[pl.ds(ro + L, L)] = plsc.load_gather(ring_v.at[s],
                                                       [lane + L, c])

            @pl.when(r + _NBUF < _BPW)
            def _():
                _fetch(r + _NBUF, s).start()
        return carry

    lax.fori_loop(0, _BPW // _NBUF, ublock, 0)

    pltpu.sync_copy(out_v, out_hbm.at[pl.ds(base * _PITCH, _BPW * _PITCH)])


@jax.jit
def _run(user_id, timestamp, utab_t, ttab_f, buckets, mean16, scale16):
    mesh = plsc.VectorSubcoreMesh(core_axis_name="c", subcore_axis_name="s")
    f = functools.partial(
        pl.kernel,
        mesh=mesh,
        out_type=jax.ShapeDtypeStruct((B * _PITCH,), jnp.float32),
        scratch_types=[
            pltpu.VMEM((_BPW + L,), jnp.int32),
            pltpu.VMEM((_BPW,), jnp.float32),
            pltpu.VMEM((_BPW,), jnp.int32),
            pltpu.VMEM((_NBUF, DIM, 128), jnp.float32),
            pltpu.VMEM((TSROWS * DIM,), jnp.float32),
            pltpu.VMEM((_BPW * _PITCH,), jnp.float32),
            pltpu.VMEM((NBUCKETS,), jnp.float32),
            pltpu.VMEM((2 * L,), jnp.float32),
            pltpu.SemaphoreType.DMA((_NBUF,)),
            pltpu.SemaphoreType.DMA,
        ],
        compiler_params=pltpu.CompilerParams(use_tc_tiling_on_sc=True,
                                             needs_layout_passes=False,
                                             disable_bounds_checks=True),
    )(_body)
    out = f(user_id, timestamp, utab_t, ttab_f, buckets, mean16, scale16)
    return out.reshape(B, _PITCH)[:, :ODIM]


def kernel(user_id, timestamp, user_table, ts_table, buckets, norm_mean,
           norm_var):
    scale = lax.rsqrt(norm_var[0] + 1e-6)
    mean16 = jnp.broadcast_to(norm_mean[0], (L,))
    scale16 = jnp.broadcast_to(scale, (L,))
    utab_t = user_table.T
    ttab_f = jnp.pad(ts_table.reshape(-1),
                     (0, (TSROWS - ts_table.shape[0]) * DIM))
    return _run(user_id.astype(jnp.int32), timestamp, utab_t, ttab_f,
                buckets, mean16, scale16)
